# baseline (speedup 1.0000x reference)
import functools

import jax
import jax.numpy as jnp
from jax import lax
from jax.experimental import pallas as pl
from jax.experimental.pallas import tpu as pltpu
from jax.experimental.pallas import tpu_sc as plsc

N = 10000
E = 320000
D = 128
R = 32
NC, NS = 2, 16
NW = NC * NS
CH = 80
CHG = 128

N_PAD = 10240
ROWS_PER_TILE = N_PAD // NS
NODE_PAD = NW * 3 * CHG
E_PAD = NW * 128 * CH
NCHUNK = E_PAD // (NW * CH)
DUMMY = N

_mesh = plsc.VectorSubcoreMesh(core_axis_name="c", subcore_axis_name="s")


def _zero_rows(buf, ncols):
    z = jnp.zeros((16,), jnp.float32)

    def row(i, _):
        for d in range(ncols // 16):
            buf[i, pl.ds(d * 16, 16)] = z
        return 0

    lax.fori_loop(0, buf.shape[0], row, 0)


@functools.partial(
    pl.kernel,
    out_type=jax.ShapeDtypeStruct((NODE_PAD, D), jnp.float32),
    mesh=_mesh,
    scratch_types=[
        pltpu.VMEM((CHG,), jnp.int32),
        pltpu.VMEM((CHG,), jnp.int32),
        pltpu.VMEM((CHG, D), jnp.float32),
        pltpu.VMEM((CHG, D), jnp.float32),
        pltpu.SemaphoreType.DMA,
        pltpu.SemaphoreType.DMA,
    ],
)
def _gather_nodes(table_hbm, nidx_hbm, out_hbm, i0, i1, r0, r1, s0, s1):
    wid = lax.axis_index("c") * NS + lax.axis_index("s")
    idx = (i0, i1)
    rows = (r0, r1)
    sems = (s0, s1)
    nch = NODE_PAD // (NW * CHG)
    base = wid * nch
    pltpu.sync_copy(nidx_hbm.at[base], i0)
    pltpu.async_copy(table_hbm.at[i0], r0, s0)
    for j in range(nch):
        b = j % 2
        if j + 1 < nch:
            b2 = (j + 1) % 2
            pltpu.sync_copy(nidx_hbm.at[base + j + 1], idx[b2])
            pltpu.async_copy(table_hbm.at[idx[b2]], rows[b2], sems[b2])
        pltpu.make_async_copy(table_hbm.at[idx[b]], rows[b], sems[b]).wait()
        pltpu.sync_copy(rows[b], out_hbm.at[pl.ds((base + j) * CHG, CHG)])


@functools.partial(
    pl.kernel,
    out_type=jax.ShapeDtypeStruct((NC, N_PAD, D), jnp.float32),
    mesh=_mesh,
    scratch_types=[
        pltpu.VMEM((CH,), jnp.int32),
        pltpu.VMEM((CH,), jnp.int32),
        pltpu.VMEM((CH, D), jnp.float32),
        pltpu.VMEM_SHARED((N_PAD, D), jnp.float32),
        pltpu.SemaphoreType.DMA,
        pltpu.SemaphoreType.DMA,
    ],
)
def _deg_count(idx_hbm, out_hbm, d0, d1, ones, deg_s, s0, s1):
    c = lax.axis_index("c")
    s = lax.axis_index("s")
    wid = c * NS + s
    base = s * ROWS_PER_TILE

    _zero_rows(ones, D)
    for k in range(ROWS_PER_TILE // CH):
        pltpu.sync_copy(ones, deg_s.at[pl.ds(base + k * CH, CH)])
    o = jnp.full((16,), 1.0, jnp.float32)

    def onesrow(i, _):
        ones[i, pl.ds(0, 16)] = o
        return 0

    lax.fori_loop(0, CH, onesrow, 0)
    plsc.subcore_barrier()

    didx = (d0, d1)
    sems = (s0, s1)
    row0 = wid * NCHUNK

    def pair(p, _):
        for b in range(2):
            t = 2 * p + b

            @pl.when(p >= 1)
            def _():
                pltpu.make_async_copy(ones, deg_s.at[didx[b]], sems[b]).wait()

            pltpu.sync_copy(idx_hbm.at[row0 + t, 1], didx[b])
            pltpu.async_copy(ones, deg_s.at[didx[b]], sems[b], add=True)
        return 0

    lax.fori_loop(0, NCHUNK // 2, pair, 0)
    for b in range(2):
        pltpu.make_async_copy(ones, deg_s.at[didx[b]], sems[b]).wait()
    plsc.subcore_barrier()

    for k in range(ROWS_PER_TILE // CH):
        sl = pl.ds(base + k * CH, CH)
        pltpu.sync_copy(deg_s.at[sl], out_hbm.at[c, sl])


@functools.partial(
    pl.kernel,
    out_type=jax.ShapeDtypeStruct((NC, N_PAD, D), jnp.float32),
    mesh=_mesh,
    scratch_types=[
        pltpu.VMEM((3, CH), jnp.int32),
        pltpu.VMEM((3, CH), jnp.int32),
        pltpu.VMEM((3, CH), jnp.int32),
        pltpu.VMEM((3, CH), jnp.int32),
        pltpu.VMEM((CH, D), jnp.float32),
        pltpu.VMEM((CH, D), jnp.float32),
        pltpu.VMEM((CH, D), jnp.float32),
        pltpu.VMEM((CH, D), jnp.float32),
        pltpu.VMEM_SHARED((R, D), jnp.float32),
        pltpu.VMEM_SHARED((N_PAD, D), jnp.float32),
        pltpu.SemaphoreType.DMA,
        pltpu.SemaphoreType.DMA,
        pltpu.SemaphoreType.DMA,
        pltpu.SemaphoreType.DMA,
        pltpu.SemaphoreType.DMA,
        pltpu.SemaphoreType.DMA,
        pltpu.SemaphoreType.DMA,
        pltpu.SemaphoreType.DMA,
    ],
)
def _edge_agg(h_hbm, rel_hbm, idx_hbm, agg_out,
              ib0, ib1, ib2, ib3, hb0, hb1, wb0, wb1, rel_s, agg_s,
              hs0, hs1, ws0, ws1, ss0, ss1, is0, is1):
    c = lax.axis_index("c")
    s = lax.axis_index("s")
    wid = c * NS + s
    base = s * ROWS_PER_TILE

    _zero_rows(hb0, D)
    for k in range(ROWS_PER_TILE // CH):
        pltpu.sync_copy(hb0, agg_s.at[pl.ds(base + k * CH, CH)])

    @pl.when(s == 0)
    def _():
        pltpu.sync_copy(rel_hbm, rel_s)

    plsc.subcore_barrier()

    ib = (ib0, ib1, ib2, ib3)
    hb = (hb0, hb1)
    wb = (wb0, wb1)
    hs = (hs0, hs1)
    ws = (ws0, ws1)
    ss = (ss0, ss1)
    isem = (is0, is1)
    row0 = wid * NCHUNK

    pltpu.sync_copy(idx_hbm.at[row0], ib0)
    pltpu.async_copy(idx_hbm.at[row0 + 1], ib1, is1)
    pltpu.async_copy(h_hbm.at[ib0.at[0]], hb0, hs0)
    pltpu.async_copy(rel_s.at[ib0.at[2]], wb0, ws0)

    def quad(q, _):
        for b in range(4):
            t = 4 * q + b
            p0 = b % 2
            p1 = (b + 1) % 2
            tb = ib[b]
            tb1 = ib[(b + 1) % 4]
            tbm1 = ib[(b + 3) % 4]
            pltpu.make_async_copy(h_hbm.at[tb.at[0]], hb[p0], hs[p0]).wait()
            pltpu.make_async_copy(rel_s.at[tb.at[2]], wb[p0], ws[p0]).wait()

            @pl.when(t >= 1)
            def _():
                pltpu.make_async_copy(hb[p1], agg_s.at[tbm1.at[1]], ss[p1]).wait()

            @pl.when(t + 2 < NCHUNK)
            def _():
                pltpu.async_copy(idx_hbm.at[row0 + t + 2], ib[(b + 2) % 4],
                                 isem[p0])

            @pl.when(t + 1 < NCHUNK)
            def _():
                pltpu.make_async_copy(idx_hbm.at[row0 + t + 1], tb1,
                                      isem[p1]).wait()
                pltpu.async_copy(h_hbm.at[tb1.at[0]], hb[p1], hs[p1])
                pltpu.async_copy(rel_s.at[tb1.at[2]], wb[p1], ws[p1])

            @plsc.parallel_loop(0, CH, unroll=4)
            def _(i):
                for d in range(D // 16):
                    sl = pl.ds(d * 16, 16)
                    hb[p0][i, sl] = hb[p0][i, sl] * wb[p0][i, sl]

            pltpu.async_copy(hb[p0], agg_s.at[tb.at[1]], ss[p0], add=True)
        return 0

    lax.fori_loop(0, NCHUNK // 4, quad, 0)
    pltpu.make_async_copy(hb1, agg_s.at[ib3.at[1]], ss1).wait()
    plsc.subcore_barrier()

    for k in range(ROWS_PER_TILE // CH):
        sl = pl.ds(base + k * CH, CH)
        pltpu.sync_copy(agg_s.at[sl], agg_out.at[c, sl])


N_BLK = 2000


def _make_mm_body(relu):
    def body(agg_ref, deg_ref, h_ref, ws_ref, wn_ref, b_ref, o_ref):
        agg = agg_ref[0] + agg_ref[1]
        d = deg_ref[0, :, :1] + deg_ref[1, :, :1]
        hn = agg / jnp.maximum(d, 1.0)
        acc = jnp.dot(h_ref[...], ws_ref[...], preferred_element_type=jnp.float32)
        acc += jnp.dot(hn, wn_ref[...], preferred_element_type=jnp.float32)
        acc += b_ref[...]
        o_ref[...] = jnp.maximum(acc, 0.0) if relu else acc

    return body


def _sage_mm(aggp, degp, h, Ws, Wn, b, relu):
    return pl.pallas_call(
        _make_mm_body(relu),
        grid=(N // N_BLK,),
        in_specs=[
            pl.BlockSpec((NC, N_BLK, D), lambda i: (0, i, 0)),
            pl.BlockSpec((NC, N_BLK, 8), lambda i: (0, i, 0)),
            pl.BlockSpec((N_BLK, D), lambda i: (i, 0)),
            pl.BlockSpec((D, D), lambda i: (0, 0)),
            pl.BlockSpec((D, D), lambda i: (0, 0)),
            pl.BlockSpec((1, D), lambda i: (0, 0)),
        ],
        out_specs=pl.BlockSpec((N_BLK, D), lambda i: (i, 0)),
        out_shape=jax.ShapeDtypeStruct((N, D), jnp.float32),
    )(aggp, degp, h, Ws, Wn, b.reshape(1, D))


@jax.jit
def _run(node_idx, src, dst, etype, pre_embed, rel_weght,
         W_self0, W_neigh0, b0, W_self1, W_neigh1, b1):
    i32 = jnp.int32
    nidx = jnp.concatenate(
        [node_idx.astype(i32), jnp.zeros((NODE_PAD - N,), i32)]
    ).reshape(NODE_PAD // CHG, CHG)
    src_p = jnp.concatenate(
        [src.astype(i32), jnp.zeros((E_PAD - E,), i32)]
    ).reshape(E_PAD // CH, CH)
    dst_p = jnp.concatenate(
        [dst.astype(i32), jnp.full((E_PAD - E,), DUMMY, i32)]
    ).reshape(E_PAD // CH, CH)
    typ_p = jnp.concatenate(
        [etype.astype(i32), jnp.zeros((E_PAD - E,), i32)]
    ).reshape(E_PAD // CH, CH)
    idx_p = jnp.stack([src_p, dst_p, typ_p], axis=1)

    h0p = _gather_nodes(pre_embed, nidx)
    h0 = h0p[:N]

    degp = _deg_count(idx_p)[:, :, :8]
    agg0 = _edge_agg(h0, rel_weght, idx_p)
    h1 = _sage_mm(agg0, degp, h0, W_self0, W_neigh0, b0, True)

    agg1 = _edge_agg(h1, rel_weght, idx_p)
    h2 = _sage_mm(agg1, degp, h1, W_self1, W_neigh1, b1, False)
    return h2


def kernel(node_idx, edge_index, edge_type, pre_embed, rel_weght,
           W_self0, W_neigh0, b0, W_self1, W_neigh1, b1):
    return _run(node_idx, edge_index[0], edge_index[1], edge_type,
                pre_embed, rel_weght,
                W_self0, W_neigh0, b0, W_self1, W_neigh1, b1)


# speedup vs baseline: 1.2852x; 1.2852x over previous
import functools

import jax
import jax.numpy as jnp
from jax import lax
from jax.experimental import pallas as pl
from jax.experimental.pallas import tpu as pltpu
from jax.experimental.pallas import tpu_sc as plsc

N = 10000
E = 320000
D = 128
R = 32
NC, NS = 2, 16
NW = NC * NS
CH = 80
CHG = 128

N_PAD = 10240
ROWS_PER_TILE = N_PAD // NS
NODE_PAD = NW * 3 * CHG
E_PAD = NW * 126 * CH
NCHUNK = E_PAD // (NW * CH)
DUMMY = N

_mesh = plsc.VectorSubcoreMesh(core_axis_name="c", subcore_axis_name="s")


def _zero_rows(buf, ncols):
    z = jnp.zeros((16,), jnp.float32)

    def row(i, _):
        for d in range(ncols // 16):
            buf[i, pl.ds(d * 16, 16)] = z
        return 0

    lax.fori_loop(0, buf.shape[0], row, 0)


@functools.partial(
    pl.kernel,
    out_type=(
        jax.ShapeDtypeStruct((NODE_PAD, D), jnp.float32),
        jax.ShapeDtypeStruct((NC, N_PAD, D), jnp.float32),
    ),
    mesh=_mesh,
    scratch_types=[
        pltpu.VMEM((CHG,), jnp.int32),
        pltpu.VMEM((CHG,), jnp.int32),
        pltpu.VMEM((CHG, D), jnp.float32),
        pltpu.VMEM((CHG, D), jnp.float32),
        pltpu.VMEM((CH,), jnp.int32),
        pltpu.VMEM((CH,), jnp.int32),
        pltpu.VMEM((CH, D), jnp.float32),
        pltpu.VMEM_SHARED((N_PAD, D), jnp.float32),
        pltpu.SemaphoreType.DMA,
        pltpu.SemaphoreType.DMA,
        pltpu.SemaphoreType.DMA,
        pltpu.SemaphoreType.DMA,
    ],
)
def _prep(table_hbm, nidx_hbm, idx_hbm, out_hbm, deg_out,
          i0, i1, r0, r1, d0, d1, ones, deg_s, g0, g1, s0, s1):
    c = lax.axis_index("c")
    s = lax.axis_index("s")
    wid = c * NS + s
    base = s * ROWS_PER_TILE

    _zero_rows(ones, D)
    for k in range(ROWS_PER_TILE // CH):
        pltpu.sync_copy(ones, deg_s.at[pl.ds(base + k * CH, CH)])
    o = jnp.full((16,), 1.0, jnp.float32)

    def onesrow(i, _):
        ones[i, pl.ds(0, 16)] = o
        return 0

    lax.fori_loop(0, CH, onesrow, 0)
    plsc.subcore_barrier()

    idx = (i0, i1)
    rows = (r0, r1)
    gsem = (g0, g1)
    nch = NODE_PAD // (NW * CHG)
    gbase = wid * nch
    pltpu.sync_copy(nidx_hbm.at[gbase], i0)
    pltpu.async_copy(table_hbm.at[i0], r0, g0)
    for j in range(nch):
        b = j % 2
        if j + 1 < nch:
            b2 = (j + 1) % 2
            pltpu.sync_copy(nidx_hbm.at[gbase + j + 1], idx[b2])
            pltpu.async_copy(table_hbm.at[idx[b2]], rows[b2], gsem[b2])
        pltpu.make_async_copy(table_hbm.at[idx[b]], rows[b], gsem[b]).wait()
        pltpu.sync_copy(rows[b], out_hbm.at[pl.ds((gbase + j) * CHG, CHG)])

    didx = (d0, d1)
    sems = (s0, s1)
    row0 = wid * NCHUNK

    def pair(p, _):
        for b in range(2):
            t = 2 * p + b

            @pl.when(p >= 1)
            def _():
                pltpu.make_async_copy(ones, deg_s.at[didx[b]], sems[b]).wait()

            pltpu.sync_copy(idx_hbm.at[row0 + t, 1], didx[b])
            pltpu.async_copy(ones, deg_s.at[didx[b]], sems[b], add=True)
        return 0

    lax.fori_loop(0, NCHUNK // 2, pair, 0)
    for b in range(2):
        pltpu.make_async_copy(ones, deg_s.at[didx[b]], sems[b]).wait()
    plsc.subcore_barrier()

    for k in range(ROWS_PER_TILE // CH):
        sl = pl.ds(base + k * CH, CH)
        pltpu.sync_copy(deg_s.at[sl], deg_out.at[c, sl])


@functools.partial(
    pl.kernel,
    out_type=jax.ShapeDtypeStruct((NC, N_PAD, D), jnp.float32),
    mesh=_mesh,
    scratch_types=[
        pltpu.VMEM((3, CH), jnp.int32),
        pltpu.VMEM((3, CH), jnp.int32),
        pltpu.VMEM((CH, D), jnp.float32),
        pltpu.VMEM((CH, D), jnp.float32),
        pltpu.VMEM((CH, D), jnp.float32),
        pltpu.VMEM((CH, D), jnp.float32),
        pltpu.VMEM_SHARED((R, D), jnp.float32),
        pltpu.VMEM_SHARED((N_PAD, D), jnp.float32),
        pltpu.SemaphoreType.DMA,
        pltpu.SemaphoreType.DMA,
        pltpu.SemaphoreType.DMA,
        pltpu.SemaphoreType.DMA,
        pltpu.SemaphoreType.DMA,
        pltpu.SemaphoreType.DMA,
    ],
)
def _edge_agg(h_hbm, rel_hbm, idx_hbm, agg_out,
              ib0, ib1, hb0, hb1, wb0, wb1, rel_s, agg_s,
              hs0, hs1, ws0, ws1, ss0, ss1):
    c = lax.axis_index("c")
    s = lax.axis_index("s")
    wid = c * NS + s
    base = s * ROWS_PER_TILE

    _zero_rows(hb0, D)
    for k in range(ROWS_PER_TILE // CH):
        pltpu.sync_copy(hb0, agg_s.at[pl.ds(base + k * CH, CH)])

    @pl.when(s == 0)
    def _():
        pltpu.sync_copy(rel_hbm, rel_s)

    plsc.subcore_barrier()

    ib = (ib0, ib1)
    hb = (hb0, hb1)
    wb = (wb0, wb1)
    hs = (hs0, hs1)
    ws = (ws0, ws1)
    ss = (ss0, ss1)
    row0 = wid * NCHUNK

    pltpu.sync_copy(idx_hbm.at[row0], ib0)
    pltpu.async_copy(h_hbm.at[ib0.at[0]], hb0, hs0)
    pltpu.async_copy(rel_s.at[ib0.at[2]], wb0, ws0)

    def pair(p, _):
        for b in range(2):
            t = 2 * p + b
            b2 = 1 - b
            pltpu.make_async_copy(h_hbm.at[ib[b].at[0]], hb[b], hs[b]).wait()
            pltpu.make_async_copy(rel_s.at[ib[b].at[2]], wb[b], ws[b]).wait()

            @pl.when(t >= 1)
            def _():
                pltpu.make_async_copy(hb[b2], agg_s.at[ib[b2].at[1]], ss[b2]).wait()

            @pl.when(t + 1 < NCHUNK)
            def _():
                pltpu.sync_copy(idx_hbm.at[row0 + t + 1], ib[b2])
                pltpu.async_copy(h_hbm.at[ib[b2].at[0]], hb[b2], hs[b2])
                pltpu.async_copy(rel_s.at[ib[b2].at[2]], wb[b2], ws[b2])

            @plsc.parallel_loop(0, CH, unroll=4)
            def _(i):
                for d in range(D // 16):
                    sl = pl.ds(d * 16, 16)
                    hb[b][i, sl] = hb[b][i, sl] * wb[b][i, sl]

            pltpu.async_copy(hb[b], agg_s.at[ib[b].at[1]], ss[b], add=True)
        return 0

    lax.fori_loop(0, NCHUNK // 2, pair, 0)
    pltpu.make_async_copy(hb1, agg_s.at[ib1.at[1]], ss1).wait()
    plsc.subcore_barrier()

    for k in range(ROWS_PER_TILE // CH):
        sl = pl.ds(base + k * CH, CH)
        pltpu.sync_copy(agg_s.at[sl], agg_out.at[c, sl])


N_BLK = 2000


def _make_mm_body(relu):
    def body(agg_ref, deg_ref, h_ref, ws_ref, wn_ref, b_ref, o_ref):
        agg = agg_ref[0] + agg_ref[1]
        d = deg_ref[0, :, :1] + deg_ref[1, :, :1]
        hn = agg / jnp.maximum(d, 1.0)
        acc = jnp.dot(h_ref[...], ws_ref[...], preferred_element_type=jnp.float32)
        acc += jnp.dot(hn, wn_ref[...], preferred_element_type=jnp.float32)
        acc += b_ref[...]
        o_ref[...] = jnp.maximum(acc, 0.0) if relu else acc

    return body


def _sage_mm(aggp, degp, h, Ws, Wn, b, relu):
    return pl.pallas_call(
        _make_mm_body(relu),
        grid=(N // N_BLK,),
        in_specs=[
            pl.BlockSpec((NC, N_BLK, D), lambda i: (0, i, 0)),
            pl.BlockSpec((NC, N_BLK, 8), lambda i: (0, i, 0)),
            pl.BlockSpec((N_BLK, D), lambda i: (i, 0)),
            pl.BlockSpec((D, D), lambda i: (0, 0)),
            pl.BlockSpec((D, D), lambda i: (0, 0)),
            pl.BlockSpec((1, D), lambda i: (0, 0)),
        ],
        out_specs=pl.BlockSpec((N_BLK, D), lambda i: (i, 0)),
        out_shape=jax.ShapeDtypeStruct((N, D), jnp.float32),
    )(aggp, degp, h, Ws, Wn, b.reshape(1, D))


@jax.jit
def _run(node_idx, src, dst, etype, pre_embed, rel_weght,
         W_self0, W_neigh0, b0, W_self1, W_neigh1, b1):
    i32 = jnp.int32
    nidx = jnp.concatenate(
        [node_idx.astype(i32), jnp.zeros((NODE_PAD - N,), i32)]
    ).reshape(NODE_PAD // CHG, CHG)
    src_p = jnp.concatenate(
        [src.astype(i32), jnp.zeros((E_PAD - E,), i32)]
    ).reshape(E_PAD // CH, CH)
    dst_p = jnp.concatenate(
        [dst.astype(i32), jnp.full((E_PAD - E,), DUMMY, i32)]
    ).reshape(E_PAD // CH, CH)
    typ_p = jnp.concatenate(
        [etype.astype(i32), jnp.zeros((E_PAD - E,), i32)]
    ).reshape(E_PAD // CH, CH)
    idx_p = jnp.stack([src_p, dst_p, typ_p], axis=1)

    h0p, degf = _prep(pre_embed, nidx, idx_p)
    h0 = h0p[:N]
    degp = degf[:, :, :8]

    agg0 = _edge_agg(h0, rel_weght, idx_p)
    h1 = _sage_mm(agg0, degp, h0, W_self0, W_neigh0, b0, True)

    agg1 = _edge_agg(h1, rel_weght, idx_p)
    h2 = _sage_mm(agg1, degp, h1, W_self1, W_neigh1, b1, False)
    return h2


def kernel(node_idx, edge_index, edge_type, pre_embed, rel_weght,
           W_self0, W_neigh0, b0, W_self1, W_neigh1, b1):
    return _run(node_idx, edge_index[0], edge_index[1], edge_type,
                pre_embed, rel_weght,
                W_self0, W_neigh0, b0, W_self1, W_neigh1, b1)
